# baseline (speedup 1.0000x reference)
import functools

import jax
import jax.numpy as jnp
from jax import lax
from jax.experimental import pallas as pl
from jax.experimental.pallas import tpu as pltpu
from jax.experimental.pallas import tpu_sc as plsc

EMBED_DIM = 64
BATCH = 16384
NUM_CORES = 2
NUM_SUBCORES = 16
NUM_WORKERS = NUM_CORES * NUM_SUBCORES
B_PER_W = BATCH // NUM_WORKERS
CHUNK = 128
N_CHUNKS = B_PER_W // CHUNK


def _gather_body(idx_hbm, table_hbm, out_hbm, idx_v, rows_v, sem):
    wid = lax.axis_index("s") * NUM_CORES + lax.axis_index("c")
    pltpu.sync_copy(idx_hbm.at[wid], idx_v)
    copies = [
        pltpu.make_async_copy(table_hbm.at[idx_v.at[j]], rows_v.at[j], sem)
        for j in range(N_CHUNKS)
    ]
    for c in copies:
        c.start()
    for c in copies:
        c.wait()
    pltpu.sync_copy(rows_v, out_hbm.at[wid])


@functools.partial(
    pl.kernel,
    out_type=jax.ShapeDtypeStruct(
        (NUM_WORKERS, N_CHUNKS, CHUNK, EMBED_DIM), jnp.float32
    ),
    mesh=plsc.VectorSubcoreMesh(core_axis_name="c", subcore_axis_name="s"),
    scratch_types=[
        pltpu.VMEM((N_CHUNKS, CHUNK), jnp.int32),
        pltpu.VMEM((N_CHUNKS, CHUNK, EMBED_DIM), jnp.float32),
        pltpu.SemaphoreType.DMA,
    ],
    compiler_params=pltpu.CompilerParams(use_tc_tiling_on_sc=False),
)
def _gather_kernel(idx_hbm, table_hbm, out_hbm, idx_v, rows_v, sem):
    _gather_body(idx_hbm, table_hbm, out_hbm, idx_v, rows_v, sem)


def kernel(input_, weight):
    idx = input_.astype(jnp.int32).reshape(NUM_WORKERS, N_CHUNKS, CHUNK)
    out = _gather_kernel(idx, weight)
    return out.reshape(BATCH, EMBED_DIM)


# speedup vs baseline: 1.6508x; 1.6508x over previous
import functools

import jax
import jax.numpy as jnp
from jax import lax
from jax.experimental import pallas as pl
from jax.experimental.pallas import tpu as pltpu
from jax.experimental.pallas import tpu_sc as plsc

EMBED_DIM = 64
BATCH = 16384
NUM_CORES = 2
NUM_SUBCORES = 16
NUM_WORKERS = NUM_CORES * NUM_SUBCORES
B_PER_W = BATCH // NUM_WORKERS
CHUNK = 32
N_CHUNKS = B_PER_W // CHUNK


def _gather_body(idx_hbm, table_hbm, out_hbm, idx_v, rows_v, sem):
    wid = lax.axis_index("s") * NUM_CORES + lax.axis_index("c")
    pltpu.sync_copy(idx_hbm.at[wid], idx_v)
    for c in range(N_CHUNKS):
        copies = []
        for g in range(CHUNK // 16):
            vec = idx_v[pl.ds(c * CHUNK + g * 16, 16)]
            for k in range(16):
                copies.append(
                    pltpu.make_async_copy(
                        table_hbm.at[pl.ds(vec[k], 1)],
                        rows_v.at[pl.ds(c * CHUNK + g * 16 + k, 1)],
                        sem,
                    )
                )
        for cp in copies:
            cp.start()
        for cp in copies:
            cp.wait()
    pltpu.sync_copy(rows_v, out_hbm.at[wid])


@functools.partial(
    pl.kernel,
    out_type=jax.ShapeDtypeStruct(
        (NUM_WORKERS, B_PER_W, EMBED_DIM), jnp.float32
    ),
    mesh=plsc.VectorSubcoreMesh(core_axis_name="c", subcore_axis_name="s"),
    scratch_types=[
        pltpu.VMEM((B_PER_W,), jnp.int32),
        pltpu.VMEM((B_PER_W, EMBED_DIM), jnp.float32),
        pltpu.SemaphoreType.DMA,
    ],
)
def _gather_kernel(idx_hbm, table_hbm, out_hbm, idx_v, rows_v, sem):
    _gather_body(idx_hbm, table_hbm, out_hbm, idx_v, rows_v, sem)


def kernel(input_, weight):
    idx = input_.astype(jnp.int32).reshape(NUM_WORKERS, B_PER_W)
    out = _gather_kernel(idx, weight)
    return out.reshape(BATCH, EMBED_DIM)


# speedup vs baseline: 1.6790x; 1.0171x over previous
import functools

import jax
import jax.numpy as jnp
from jax import lax
from jax.experimental import pallas as pl
from jax.experimental.pallas import tpu as pltpu
from jax.experimental.pallas import tpu_sc as plsc

EMBED_DIM = 64
BATCH = 16384
NUM_CORES = 2
NUM_SUBCORES = 16
NUM_WORKERS = NUM_CORES * NUM_SUBCORES
B_PER_W = BATCH // NUM_WORKERS
N_SEMS = 8


def _gather_body(idx_hbm, table_hbm, out_hbm, idx_v, rows_v, sems):
    wid = lax.axis_index("s") * NUM_CORES + lax.axis_index("c")
    pltpu.sync_copy(idx_hbm.at[wid], idx_v)
    copies = []
    for g in range(B_PER_W // 16):
        vec = idx_v[pl.ds(g * 16, 16)]
        for k in range(16):
            j = g * 16 + k
            copies.append(
                pltpu.make_async_copy(
                    table_hbm.at[pl.ds(vec[k], 1)],
                    rows_v.at[pl.ds(j, 1)],
                    sems[j % N_SEMS],
                )
            )
    for cp in copies:
        cp.start()
    for cp in copies:
        cp.wait()
    pltpu.sync_copy(rows_v, out_hbm.at[wid])


@functools.partial(
    pl.kernel,
    out_type=jax.ShapeDtypeStruct(
        (NUM_WORKERS, B_PER_W, EMBED_DIM), jnp.float32
    ),
    mesh=plsc.VectorSubcoreMesh(core_axis_name="c", subcore_axis_name="s"),
    scratch_types=[
        pltpu.VMEM((B_PER_W,), jnp.int32),
        pltpu.VMEM((B_PER_W, EMBED_DIM), jnp.float32),
        [pltpu.SemaphoreType.DMA] * N_SEMS,
    ],
)
def _gather_kernel(idx_hbm, table_hbm, out_hbm, idx_v, rows_v, sems):
    _gather_body(idx_hbm, table_hbm, out_hbm, idx_v, rows_v, sems)


def kernel(input_, weight):
    idx = input_.astype(jnp.int32).reshape(NUM_WORKERS, B_PER_W)
    out = _gather_kernel(idx, weight)
    return out.reshape(BATCH, EMBED_DIM)
